# baseline (speedup 1.0000x reference)
import jax
import jax.numpy as jnp
from jax import lax
from jax.experimental import pallas as pl
from jax.experimental.pallas import tpu as pltpu
from jax.experimental.pallas import tpu_sc as plsc

NUM_USERS = 100000
VOCAB = 1000
PK = VOCAB // 2
BATCH = 4096
GAIN = 50.0
CLIP = 2000.0

_L = 16
_NW = 32
_BPW = BATCH // _NW
_Q = 4
_NB = _BPW // 8
_NSLICE = PK // _L


def _sc_body(uid_hbm, w_hbm, out_hbm, idx_v, band_v, lane_v, in0, in1,
             out_buf, gs0, gs1):
    wid = lax.axis_index("s") * 2 + lax.axis_index("c")
    base = wid * _BPW
    pltpu.sync_copy(uid_hbm.at[pl.ds(base, _BPW)], idx_v)
    for j in range(_BPW // _L):
        ids = idx_v[pl.ds(j * _L, _L)]
        band_v[pl.ds(j * _L, _L)] = lax.shift_right_logical(ids, 3)
        lane_v[pl.ds(j * _L, _L)] = lax.bitwise_and(ids, 7)

    in_bufs = (in0, in1)
    gsems = (gs0, gs1)

    def fire(q, slot):
        for k in range(_Q):
            row0 = pl.multiple_of(band_v[pl.ds(q * _Q + k, _L)][0] * 8, 8)
            pltpu.async_copy(w_hbm.at[pl.ds(row0, 8)],
                             in_bufs[slot].at[k], gsems[slot])

    def drain(slot):
        for k in range(_Q):
            pltpu.make_async_copy(w_hbm.at[pl.ds(0, 8)],
                                  in_bufs[slot].at[k], gsems[slot]).wait()

    def compute(q, slot, half):
        in_buf = in_bufs[slot]

        def do_row(k, carry):
            lane = lane_v[pl.ds(q * _Q + k, _L)][0]
            for j in range(_NSLICE):
                out_buf[half + k, pl.ds(j * _L, _L)] = in_buf[
                    k, lane, pl.ds(j * _L, _L)]
            out_buf[half + k, pl.ds(PK - _L, _L)] = in_buf[
                k, lane, pl.ds(PK - _L, _L)]
            return carry

        lax.fori_loop(0, _Q, do_row, 0)

    fire(0, 0)
    fire(1, 1)

    def do_band(s, carry):
        q0 = s * 2

        drain(0)
        compute(q0, 0, 0)

        @pl.when(s < _NB - 1)
        def _():
            fire(q0 + 2, 0)

        drain(1)
        compute(q0 + 1, 1, _Q)

        @pl.when(s < _NB - 1)
        def _():
            fire(q0 + 3, 1)

        pltpu.sync_copy(
            out_buf, out_hbm.at[pl.ds(pl.multiple_of(base + s * 8, 8), 8)])
        return carry

    lax.fori_loop(0, _NB, do_band, 0)


def kernel(user_ids, weight):
    mesh = plsc.VectorSubcoreMesh(core_axis_name="c", subcore_axis_name="s")
    f = pl.kernel(
        _sc_body,
        mesh=mesh,
        out_type=jax.ShapeDtypeStruct((BATCH, PK), jnp.float32),
        scratch_types=[
            pltpu.VMEM((_BPW,), jnp.int32),
            pltpu.VMEM((_BPW + _L,), jnp.int32),
            pltpu.VMEM((_BPW + _L,), jnp.int32),
            pltpu.VMEM((_Q, 8, PK), jnp.float32),
            pltpu.VMEM((_Q, 8, PK), jnp.float32),
            pltpu.VMEM((8, PK), jnp.float32),
            pltpu.SemaphoreType.DMA,
            pltpu.SemaphoreType.DMA,
        ],
    )
    wb = jnp.clip(weight * GAIN, -CLIP, CLIP).astype(jnp.bfloat16)
    wpk = lax.bitcast_convert_type(
        wb.reshape(NUM_USERS, PK, 2), jnp.float32)
    o = f(user_ids.astype(jnp.int32), wpk)
    ob = lax.bitcast_convert_type(
        o.reshape(BATCH, PK, 1), jnp.bfloat16).reshape(BATCH, VOCAB)
    return ob.astype(jnp.float32)


# speedup vs baseline: 4.2426x; 4.2426x over previous
import jax
import jax.numpy as jnp
from jax import lax
from jax.experimental import pallas as pl
from jax.experimental.pallas import tpu as pltpu
from jax.experimental.pallas import tpu_sc as plsc

NUM_USERS = 100000
VOCAB = 1000
BATCH = 4096
GAIN = 50.0
CLIP = 2000.0

_L = 16
_NW = 32
_BPW = BATCH // _NW
_Q = 4
_NB = _BPW // 8
_NSLICE = VOCAB // _L


def _sc_body(uid_hbm, w_hbm, out_hbm, idx_v, band_v, lane_v, in0, in1,
             out_buf, gs0, gs1):
    wid = lax.axis_index("s") * 2 + lax.axis_index("c")
    base = wid * _BPW
    pltpu.sync_copy(uid_hbm.at[pl.ds(base, _BPW)], idx_v)
    for j in range(_BPW // _L):
        ids = idx_v[pl.ds(j * _L, _L)]
        band_v[pl.ds(j * _L, _L)] = lax.shift_right_logical(ids, 3)
        lane_v[pl.ds(j * _L, _L)] = lax.bitwise_and(ids, 7)

    in_bufs = (in0, in1)
    gsems = (gs0, gs1)

    def fire(q, slot):
        for k in range(_Q):
            row0 = pl.multiple_of(band_v[pl.ds(q * _Q + k, _L)][0] * 8, 8)
            pltpu.async_copy(w_hbm.at[pl.ds(row0, 8)],
                             in_bufs[slot].at[k], gsems[slot])

    def drain(slot):
        for k in range(_Q):
            pltpu.make_async_copy(w_hbm.at[pl.ds(0, 8)],
                                  in_bufs[slot].at[k], gsems[slot]).wait()

    def compute(q, slot, half):
        in_buf = in_bufs[slot]

        def do_row(k, carry):
            lane = lane_v[pl.ds(q * _Q + k, _L)][0]
            for j in range(_NSLICE):
                x = in_buf[k, lane, pl.ds(j * _L, _L)]
                out_buf[half + k, pl.ds(j * _L, _L)] = jnp.clip(
                    x * GAIN, -CLIP, CLIP)
            x = in_buf[k, lane, pl.ds(VOCAB - _L, _L)]
            out_buf[half + k, pl.ds(VOCAB - _L, _L)] = jnp.clip(
                x * GAIN, -CLIP, CLIP)
            return carry

        lax.fori_loop(0, _Q, do_row, 0)

    fire(0, 0)
    fire(1, 1)

    def do_band(s, carry):
        q0 = s * 2

        drain(0)
        compute(q0, 0, 0)

        @pl.when(s < _NB - 1)
        def _():
            fire(q0 + 2, 0)

        drain(1)
        compute(q0 + 1, 1, _Q)

        @pl.when(s < _NB - 1)
        def _():
            fire(q0 + 3, 1)

        pltpu.sync_copy(
            out_buf, out_hbm.at[pl.ds(pl.multiple_of(base + s * 8, 8), 8)])
        return carry

    lax.fori_loop(0, _NB, do_band, 0)


def kernel(user_ids, weight):
    mesh = plsc.VectorSubcoreMesh(core_axis_name="c", subcore_axis_name="s")
    f = pl.kernel(
        _sc_body,
        mesh=mesh,
        out_type=jax.ShapeDtypeStruct((BATCH, VOCAB), jnp.float32),
        scratch_types=[
            pltpu.VMEM((_BPW,), jnp.int32),
            pltpu.VMEM((_BPW + _L,), jnp.int32),
            pltpu.VMEM((_BPW + _L,), jnp.int32),
            pltpu.VMEM((_Q, 8, VOCAB), jnp.float32),
            pltpu.VMEM((_Q, 8, VOCAB), jnp.float32),
            pltpu.VMEM((8, VOCAB), jnp.float32),
            pltpu.SemaphoreType.DMA,
            pltpu.SemaphoreType.DMA,
        ],
    )
    return f(user_ids.astype(jnp.int32), weight)
